# baseline (speedup 1.0000x reference)
import jax
import jax.numpy as jnp
from jax.experimental import pallas as pl

_N_TOK = 16384
_D = 2048
_E = 16
_BT = 1024


def _gating_body(x_ref, wt_ref, b_ref, n_ref, o_ref):
    logits = jnp.dot(x_ref[...], wt_ref[...],
                     preferred_element_type=jnp.float32)
    nl = logits + b_ref[...] + n_ref[...]

    e = jax.lax.broadcasted_iota(jnp.int32, nl.shape, 1)
    m1 = jnp.max(nl, axis=1, keepdims=True)
    i1 = jnp.min(jnp.where(nl == m1, e, _E), axis=1, keepdims=True)
    mask1 = e == i1
    nl2 = jnp.where(mask1, -jnp.inf, nl)
    m2 = jnp.max(nl2, axis=1, keepdims=True)
    i2 = jnp.min(jnp.where(nl2 == m2, e, _E), axis=1, keepdims=True)
    mask2 = e == i2

    t = jnp.exp(m2 - m1)
    w1 = 1.0 / (1.0 + t)
    w2 = t * w1
    o_ref[...] = jnp.where(mask1, w1, jnp.where(mask2, w2, 0.0))


def kernel(x, W, b):
    n_tok, d = x.shape
    noise = jax.random.normal(jax.random.key(42), (n_tok, _E),
                              dtype=jnp.float32) * 0.1
    wt = W.T
    b_row = b[None, :]
    grid = (n_tok // _BT,)
    return pl.pallas_call(
        _gating_body,
        grid=grid,
        in_specs=[
            pl.BlockSpec((_BT, d), lambda i: (i, 0)),
            pl.BlockSpec((d, _E), lambda i: (0, 0)),
            pl.BlockSpec((1, _E), lambda i: (0, 0)),
            pl.BlockSpec((_BT, _E), lambda i: (i, 0)),
        ],
        out_specs=pl.BlockSpec((_BT, _E), lambda i: (i, 0)),
        out_shape=jax.ShapeDtypeStruct((n_tok, _E), jnp.float32),
    )(x, wt, b_row, noise)


# speedup vs baseline: 1.0088x; 1.0088x over previous
import jax
import jax.numpy as jnp
from jax.experimental import pallas as pl

_N_TOK = 16384
_D = 2048
_E = 16
_BT = 1024


def _gating_body(x_ref, wt_ref, nb_ref, o_ref):
    xh = x_ref[...].astype(jnp.bfloat16)
    logits = jnp.dot(xh, wt_ref[...], preferred_element_type=jnp.float32)
    nl = logits + nb_ref[...]

    m1 = jnp.max(nl, axis=1, keepdims=True)
    mask1 = nl == m1
    nl2 = jnp.where(mask1, -jnp.inf, nl)
    m2 = jnp.max(nl2, axis=1, keepdims=True)
    mask2 = nl2 == m2

    t = jnp.exp(m2 - m1)
    w1 = 1.0 / (1.0 + t)
    w2 = t * w1
    o_ref[...] = jnp.where(mask1, w1, jnp.where(mask2, w2, 0.0))


def kernel(x, W, b):
    n_tok, d = x.shape
    noise = jax.random.normal(jax.random.key(42), (n_tok, _E),
                              dtype=jnp.float32) * 0.1
    nb = noise + b[None, :]
    wt = W.T.astype(jnp.bfloat16)
    grid = (n_tok // _BT,)
    return pl.pallas_call(
        _gating_body,
        grid=grid,
        in_specs=[
            pl.BlockSpec((_BT, d), lambda i: (i, 0)),
            pl.BlockSpec((d, _E), lambda i: (0, 0)),
            pl.BlockSpec((_BT, _E), lambda i: (i, 0)),
        ],
        out_specs=pl.BlockSpec((_BT, _E), lambda i: (i, 0)),
        out_shape=jax.ShapeDtypeStruct((n_tok, _E), jnp.float32),
    )(x, wt, nb)


# speedup vs baseline: 1.0210x; 1.0121x over previous
import jax
import jax.numpy as jnp
from jax.experimental import pallas as pl

_N_TOK = 16384
_D = 2048
_E = 16
_BT = 1024


def _gating_body(x_ref, wt_ref, nb_ref, o_ref):
    i = pl.program_id(0)
    xh = x_ref[...].astype(jnp.bfloat16)
    logits = jnp.dot(xh, wt_ref[...], preferred_element_type=jnp.float32)
    nl = logits + nb_ref[pl.ds(i * _BT, _BT), :]

    m1 = jnp.max(nl, axis=1, keepdims=True)
    mask1 = nl == m1
    nl2 = jnp.where(mask1, -jnp.inf, nl)
    m2 = jnp.max(nl2, axis=1, keepdims=True)
    mask2 = nl2 == m2

    t = jnp.exp(m2 - m1)
    w1 = 1.0 / (1.0 + t)
    w2 = t * w1
    o_ref[...] = jnp.where(mask1, w1, jnp.where(mask2, w2, 0.0))


def kernel(x, W, b):
    n_tok, d = x.shape
    noise = jax.random.normal(jax.random.key(42), (n_tok, _E),
                              dtype=jnp.float32) * 0.1
    nb = noise + b[None, :]
    wt = W.T.astype(jnp.bfloat16)
    grid = (n_tok // _BT,)
    return pl.pallas_call(
        _gating_body,
        grid=grid,
        in_specs=[
            pl.BlockSpec((_BT, d), lambda i: (i, 0)),
            pl.BlockSpec((d, _E), lambda i: (0, 0)),
            pl.BlockSpec((n_tok, _E), lambda i: (0, 0)),
        ],
        out_specs=pl.BlockSpec((_BT, _E), lambda i: (i, 0)),
        out_shape=jax.ShapeDtypeStruct((n_tok, _E), jnp.float32),
    )(x, wt, nb)
